# Initial kernel scaffold; baseline (speedup 1.0000x reference)
import jax
import jax.numpy as jnp
from jax.experimental import pallas as pl


def kernel(x_v, x_e, v_up_index, v_up_edge, e_down_index, e_down_vert, e_up_index, batch_v, batch_e, params):
    raise NotImplementedError("write your pallas kernel here")


# speedup vs baseline: 1.5952x; 1.5952x over previous
import functools

import jax
import jax.numpy as jnp
from jax.experimental import pallas as pl

D = 128
EPS = 1e-5


def _dense_body(x_ref, w_ref, b_ref, o_ref, *, relu):
    acc = jnp.dot(x_ref[...], w_ref[...], preferred_element_type=jnp.float32)
    acc = acc + b_ref[...]
    if relu:
        acc = jnp.maximum(acc, 0.0)
    o_ref[...] = acc


@functools.partial(jax.jit, static_argnames=("relu",))
def _dense(x, w, b, relu=False):
    R, K = x.shape[0], w.shape[1]
    BR = 1024
    grid = (pl.cdiv(R, BR),)
    return pl.pallas_call(
        functools.partial(_dense_body, relu=relu),
        grid=grid,
        in_specs=[
            pl.BlockSpec((BR, x.shape[1]), lambda i: (i, 0)),
            pl.BlockSpec((x.shape[1], K), lambda i: (0, 0)),
            pl.BlockSpec((1, K), lambda i: (0, 0)),
        ],
        out_specs=pl.BlockSpec((BR, K), lambda i: (i, 0)),
        out_shape=jax.ShapeDtypeStruct((R, K), jnp.float32),
    )(x, w, b.reshape(1, K))


def _bn(x, g, b):
    m = x.mean(axis=0)
    v = x.var(axis=0)
    return (x - m) / jnp.sqrt(v + EPS) * g + b


def kernel(x_v, x_e, v_up_index, v_up_edge, e_down_index, e_down_vert,
           e_up_index, batch_v, batch_e, params):
    p = params
    src, dst = v_up_index[0], v_up_index[1]
    es, ed = e_down_index[0], e_down_index[1]
    us, ud = e_up_index[0], e_up_index[1]

    for l in range(3):
        wv = p[f"L{l}_vup_W"]
        we = p[f"L{l}_edown_W"]
        pv = _dense(x_v, jnp.concatenate([wv[:D], we[D:]], axis=1),
                    jnp.zeros((2 * D,), jnp.float32))
        pe = _dense(x_e, jnp.concatenate([wv[D:], we[:D], p[f"L{l}_eup_W"]], axis=1),
                    jnp.concatenate([jnp.zeros((2 * D,), jnp.float32), p[f"L{l}_eup_b"]]))
        Pv, Qv = pv[:, :D], pv[:, D:]
        Qe, Pe, Ru = pe[:, :D], pe[:, D:2 * D], jnp.maximum(pe[:, 2 * D:], 0.0)

        m = jnp.maximum(Pv[src] + Qe[v_up_edge] + p[f"L{l}_vup_b"], 0.0)
        m = _bn(m, p[f"L{l}_vup_g"], p[f"L{l}_vup_be"])
        agg_v = jax.ops.segment_sum(m, dst, num_segments=x_v.shape[0])

        md = jnp.maximum(Pe[es] + Qv[e_down_vert] + p[f"L{l}_edown_b"], 0.0)
        md = _bn(md, p[f"L{l}_edown_g"], p[f"L{l}_edown_be"])
        agg_d = jax.ops.segment_sum(md, ed, num_segments=x_e.shape[0])

        mu = _bn(Ru[us], p[f"L{l}_eup_g"], p[f"L{l}_eup_be"])
        agg_u = jax.ops.segment_sum(mu, ud, num_segments=x_e.shape[0])

        hv = _dense(x_v + agg_v, p[f"L{l}_vupd_W1"], p[f"L{l}_vupd_b1"], relu=True)
        hv = _dense(hv, p[f"L{l}_vupd_W2"], p[f"L{l}_vupd_b2"], relu=True)
        x_v = _bn(hv, p[f"L{l}_vupd_g"], p[f"L{l}_vupd_be"])
        he = _dense(x_e + agg_d + agg_u, p[f"L{l}_eupd_W1"], p[f"L{l}_eupd_b1"], relu=True)
        he = _dense(he, p[f"L{l}_eupd_W2"], p[f"L{l}_eupd_b2"], relu=True)
        x_e = _bn(he, p[f"L{l}_eupd_g"], p[f"L{l}_eupd_be"])

    B = 64
    cnt_v = jnp.clip(jnp.bincount(batch_v, length=B), 1).astype(jnp.float32)
    cnt_e = jnp.clip(jnp.bincount(batch_e, length=B), 1).astype(jnp.float32)
    pooled_v = jax.ops.segment_sum(x_v, batch_v, num_segments=B) / cnt_v[:, None]
    pooled_e = jax.ops.segment_sum(x_e, batch_e, num_segments=B) / cnt_e[:, None]
    x = pooled_v + pooled_e
    x = jnp.maximum(x @ p["lin1_W"] + p["lin1_b"], 0.0)
    return x @ p["lin2_W"] + p["lin2_b"]
